# Initial kernel scaffold; baseline (speedup 1.0000x reference)
import jax
import jax.numpy as jnp
from jax.experimental import pallas as pl


def kernel(embeddings, edge_indices, Ws, a_srcs, a_dsts, bs):
    raise NotImplementedError("write your pallas kernel here")


# speedup vs baseline: 1.9652x; 1.9652x over previous
import jax
import jax.numpy as jnp
from jax import lax
from jax.experimental import pallas as pl
from jax.experimental.pallas import tpu as pltpu
import jax.experimental.pallas.tpu_sc as plsc

N = 50000
D = 64
E = 800000
C = 3

BS = 2000
NB = N // BS

K = 128
NTILES = 16
EPT_CH = K * ((E + NTILES * K - 1) // (NTILES * K))
EP = EPT_CH * NTILES
NCHUNK = EPT_CH // K
NTAB = N + 16
NHALF = 16736
NACC = 16768
Q = D // 4


def _prep_body(agg_ref, b_ref, sel_ref, w_ref, as_ref, ad_ref,
               h0, h1, h2, h3, asrc_ref, adst_ref, m_ref, mx):
    i = pl.program_id(0)
    a = agg_ref[...]
    xr = jnp.concatenate([a[0], a[1], a[2], a[3]], axis=-1)
    s = sel_ref[0:1, 0:1]
    x = jnp.maximum(xr + b_ref[...], 0.0) * s + xr * (1.0 - s)
    h = jnp.dot(x, w_ref[...], preferred_element_type=jnp.float32)
    for q, href in enumerate((h0, h1, h2, h3)):
        href[...] = h[:, q * Q:(q + 1) * Q]

    av = jnp.sum(h * as_ref[...], axis=-1)
    bv = jnp.sum(h * ad_ref[...], axis=-1)
    asrc_ref[...] = av.reshape(1, 1, BS)
    adst_ref[...] = bv.reshape(1, 1, BS)

    @pl.when(i == 0)
    def _():
        mx[0] = jnp.float32(-jnp.inf)
        mx[1] = jnp.float32(-jnp.inf)

    mx[0] = jnp.maximum(mx[0], jnp.max(av))
    mx[1] = jnp.maximum(mx[1], jnp.max(bv))
    m_ref[...] = jnp.full((1, 128), jnp.maximum(0.0, mx[0] + mx[1]),
                          dtype=jnp.float32)


_tc_prep = pl.pallas_call(
    _prep_body,
    grid=(NB,),
    in_specs=[
        pl.BlockSpec((4, BS, Q), lambda i: (0, i, 0)),
        pl.BlockSpec((1, D), lambda i: (0, 0)),
        pl.BlockSpec((1, 128), lambda i: (0, 0)),
        pl.BlockSpec((D, D), lambda i: (0, 0)),
        pl.BlockSpec((1, D), lambda i: (0, 0)),
        pl.BlockSpec((1, D), lambda i: (0, 0)),
    ],
    out_specs=[pl.BlockSpec((BS, Q), lambda i: (i, 0))] * 4 + [
        pl.BlockSpec((1, 1, BS), lambda i: (i, 0, 0)),
        pl.BlockSpec((1, 1, BS), lambda i: (i, 0, 0)),
        pl.BlockSpec((1, 128), lambda i: (0, 0)),
    ],
    out_shape=[jax.ShapeDtypeStruct((N, Q), jnp.float32)] * 4 + [
        jax.ShapeDtypeStruct((NB, 1, BS), jnp.float32),
        jax.ShapeDtypeStruct((NB, 1, BS), jnp.float32),
        jax.ShapeDtypeStruct((1, 128), jnp.float32),
    ],
    scratch_shapes=[pltpu.SMEM((2,), jnp.float32)],
)


def _combine_body(g0, g1, g2, b2_ref, out_ref):
    s = jnp.zeros((BS, D), jnp.float32)
    for ref in (g0, g1, g2):
        a = ref[...]
        s = s + jnp.concatenate([a[0], a[1], a[2], a[3]], axis=-1)
    bsum = jnp.sum(b2_ref[...], axis=0)
    out_ref[...] = (s + bsum[None, :]) * (1.0 / 3.0)


_tc_combine = pl.pallas_call(
    _combine_body,
    grid=(NB,),
    in_specs=[pl.BlockSpec((4, BS, Q), lambda i: (0, i, 0))] * 3
    + [pl.BlockSpec((C, D), lambda i: (0, 0))],
    out_specs=pl.BlockSpec((BS, D), lambda i: (i, 0)),
    out_shape=jax.ShapeDtypeStruct((N, D), jnp.float32),
)


def _sc_body(src_hbm, dst_hbm, asrc_hbm, adst_hbm, mvec_hbm, hq_hbm,
             out_flat, exs,
             tabA, tabB, mv, srcv, dstv, exv, alv, rows, stagec, ibuf,
             acc, den, sem):
    cid = lax.axis_index("c")
    sid = lax.axis_index("s")
    tb = sid * EPT_CH
    lane16 = lax.iota(jnp.int32, 16)

    zero16 = jnp.zeros((16,), jnp.float32)
    for j in range(8):
        exv[pl.ds(j * 16, 16)] = zero16
    for j in range(K):
        stagec[pl.ds(j * 16, 16)] = zero16

    def zden(k, _):
        pltpu.sync_copy(exv, den.at[pl.ds(sid * 3200 + k * K, K)])
        return 0

    lax.fori_loop(0, 25, zden, 0)

    pltpu.sync_copy(asrc_hbm, tabA)
    pltpu.sync_copy(adst_hbm, tabB)
    pltpu.sync_copy(mvec_hbm, mv)
    mvv = mv[...]

    plsc.subcore_barrier()

    def stepA(k, _):
        base = tb + k * K
        pltpu.sync_copy(src_hbm.at[pl.ds(base, K)], srcv)
        pltpu.sync_copy(dst_hbm.at[pl.ds(base, K)], dstv)
        for j in range(8):
            s16 = srcv[pl.ds(j * 16, 16)]
            d16 = dstv[pl.ds(j * 16, 16)]
            e = plsc.load_gather(tabA, [s16]) + plsc.load_gather(tabB, [d16])
            e = jnp.where(e > 0, e, 0.2 * e)
            exv[pl.ds(j * 16, 16)] = jnp.exp(e - mvv)
        pltpu.sync_copy(exv, exs.at[pl.ds(cid * EP + base, K)])
        pltpu.sync_copy(exv, den.at[dstv], add=True)
        return 0

    lax.fori_loop(0, NCHUNK, stepA, 0)

    plsc.subcore_barrier()

    pltpu.sync_copy(den.at[pl.ds(0, NTAB)], tabB)

    def bpass(q4, _):
        rhalf = q4 % 3
        p = q4 // 3
        qoff = (2 * cid + p) * N
        rbase = rhalf * NHALF

        for j in range(K):
            stagec[pl.ds(j * 16, 16)] = zero16

        def zacc(k, _):
            pltpu.sync_copy(stagec.at[pl.ds(0, 1048)],
                            acc.at[pl.ds(sid * 16768 + k * 1048, 1048)])
            return 0

        lax.fori_loop(0, 16, zacc, 0)
        plsc.subcore_barrier()

        def stepB(k, _):
            base = tb + k * K
            pltpu.sync_copy(src_hbm.at[pl.ds(base, K)], srcv)
            pltpu.sync_copy(dst_hbm.at[pl.ds(base, K)], dstv)
            pltpu.sync_copy(exs.at[pl.ds(cid * EP + base, K)], exv)
            for j in range(8):
                d16 = dstv[pl.ds(j * 16, 16)]
                dn = plsc.load_gather(tabB, [d16])
                alv[pl.ds(j * 16, 16)] = exv[pl.ds(j * 16, 16)] / (dn + 1e-16)
                srcv[pl.ds(j * 16, 16)] = srcv[pl.ds(j * 16, 16)] + qoff
                l16 = d16 - rbase
                ok = jnp.logical_and(l16 >= 0, l16 < NHALF)
                dead = NHALF + jnp.bitwise_and(d16, 31)
                dstv[pl.ds(j * 16, 16)] = jnp.where(ok, l16, dead) * Q
            pltpu.async_copy(hq_hbm.at[srcv], rows, sem).wait()
            for g in range(8):
                a16 = alv[pl.ds(g * 16, 16)]
                b16 = dstv[pl.ds(g * 16, 16)]
                for c in range(Q):
                    ibuf[c, pl.ds(g * 16, 16)] = b16 + c
                for l in range(16):
                    r = g * 16 + l
                    rv = rows[r, pl.ds(0, 16)] * a16[l]
                    plsc.store_scatter(stagec, [lane16 * K + r], rv)
            for c in range(Q):
                pltpu.sync_copy(stagec.at[pl.ds(c * K, K)],
                                acc.at[ibuf.at[c]], add=True)
            return 0

        lax.fori_loop(0, NCHUNK, stepB, 0)
        plsc.subcore_barrier()

        @pl.when(rhalf < 2)
        def _():
            pltpu.sync_copy(
                acc.at[pl.ds(sid * 16736, 16736)],
                out_flat.at[pl.ds(qoff * Q + rhalf * (NHALF * Q)
                                  + sid * 16736, 16736)])

        @pl.when(rhalf == 2)
        def _():
            pltpu.sync_copy(
                acc.at[pl.ds(sid * 16528, 16528)],
                out_flat.at[pl.ds(qoff * Q + 2 * (NHALF * Q)
                                  + sid * 16528, 16528)])

        plsc.subcore_barrier()
        return 0

    lax.fori_loop(0, 6, bpass, 0)


_sc_gat = pl.kernel(
    _sc_body,
    out_type=[
        jax.ShapeDtypeStruct((4 * N * Q,), jnp.float32),
        jax.ShapeDtypeStruct((2 * EP,), jnp.float32),
    ],
    mesh=plsc.VectorSubcoreMesh(core_axis_name="c", subcore_axis_name="s"),
    compiler_params=pltpu.CompilerParams(
        needs_layout_passes=False, use_tc_tiling_on_sc=False),
    scratch_types=[
        pltpu.VMEM((N,), jnp.float32),
        pltpu.VMEM((NTAB,), jnp.float32),
        pltpu.VMEM((16,), jnp.float32),
        pltpu.VMEM((K,), jnp.int32),
        pltpu.VMEM((K,), jnp.int32),
        pltpu.VMEM((K,), jnp.float32),
        pltpu.VMEM((K,), jnp.float32),
        pltpu.VMEM((K, Q), jnp.float32),
        pltpu.VMEM((K * Q,), jnp.float32),
        pltpu.VMEM((Q, K), jnp.int32),
        pltpu.VMEM_SHARED((NACC * Q,), jnp.float32),
        pltpu.VMEM_SHARED((51200,), jnp.float32),
        pltpu.SemaphoreType.DMA,
    ],
)


def kernel(embeddings, edge_indices, Ws, a_srcs, a_dsts, bs):
    emb_q = embeddings.reshape(N, 4, Q).transpose(1, 0, 2)
    pad_src = jnp.zeros((C, 2, EP - E), jnp.int32)
    pad_dst = jnp.full((C, 2, EP - E), N, jnp.int32)
    srcs = jnp.concatenate([edge_indices[:, :, 0], pad_src], axis=-1)
    dsts = jnp.concatenate([edge_indices[:, :, 1], pad_dst], axis=-1)
    sels = jnp.broadcast_to(
        jnp.array([0.0, 1.0], jnp.float32)[None, :, None, None], (C, 2, 1, 128))
    bprev = jnp.stack([jnp.zeros_like(bs[:, 0]), bs[:, 0]], axis=1)
    sentinel = jnp.full((NTAB - N,), -1e30, jnp.float32)

    def layer_body(aggq, xs):
        w, a_s, a_d, bb, sel, src, dst = xs
        h0, h1, h2, h3, asrc3, adst3, mvec = _tc_prep(
            aggq, bb.reshape(1, D), sel, w,
            a_s.reshape(1, D), a_d.reshape(1, D))
        hq = jnp.concatenate([h0, h1, h2, h3], axis=0)
        adst = jnp.concatenate([adst3.reshape(N), sentinel])
        out_all, _ = _sc_gat(src, dst, asrc3.reshape(N), adst,
                             mvec[0, :16], hq)
        return out_all.reshape(4, N, Q), 0

    def chan_body(carry, xs):
        aggq, _ = lax.scan(layer_body, emb_q, xs)
        return carry, aggq

    _, finals = lax.scan(
        chan_body, 0, (Ws, a_srcs, a_dsts, bprev, sels, srcs, dsts))
    return _tc_combine(finals[0], finals[1], finals[2], bs[:, 1, :])
